# baseline (speedup 1.0000x reference)
import functools

import jax
import jax.numpy as jnp
import numpy as np
from jax import lax
from jax.experimental import pallas as pl
from jax.experimental.pallas import tpu as pltpu
from jax.experimental.pallas import tpu_sc as plsc

D_MODEL = 768
MAX_LEN = 8192
LANES = 16

NUM_CORES = 2
NUM_SUBCORES = 16
NUM_WORKERS = NUM_CORES * NUM_SUBCORES

NBUF = 4
CHUNK = 32
PE_ROWS = 32


def _sinusoidal_pos_encoding(max_len, d_model):
    pos = np.arange(max_len, dtype=np.float32)[:, None]
    i = np.arange(0, d_model, 2, dtype=np.float32)
    angle = pos / np.power(10000.0, i / float(d_model))
    pe = np.zeros((max_len, d_model), dtype=np.float32)
    pe[:, 0::2] = np.sin(angle)
    pe[:, 1::2] = np.cos(angle)
    return jnp.asarray(pe)


@functools.partial(jax.jit, static_argnames=("batch", "seq_len"))
def _embed(source_flat, pe, W_enc, batch, seq_len):
    n = source_flat.shape[0]
    s_per_worker = seq_len // NUM_WORKERS
    n_sblocks = s_per_worker // PE_ROWS
    subs = PE_ROWS // CHUNK
    n_chunks = n_sblocks * batch * subs
    rows_per_worker = n // NUM_WORKERS
    mesh = plsc.VectorSubcoreMesh(
        core_axis_name="c", subcore_axis_name="s",
        num_cores=NUM_CORES, num_subcores=NUM_SUBCORES)

    @functools.partial(
        pl.kernel,
        mesh=mesh,
        out_type=jax.ShapeDtypeStruct((n, D_MODEL), jnp.float32),
        scratch_types=[
            pltpu.VMEM((rows_per_worker,), jnp.int32),
            pltpu.VMEM((PE_ROWS, D_MODEL), jnp.float32),
            pltpu.VMEM((NBUF, CHUNK, D_MODEL), jnp.float32),
            pltpu.SemaphoreType.DMA,
            pltpu.SemaphoreType.DMA,
            pltpu.SemaphoreType.DMA((NBUF,)),
            pltpu.SemaphoreType.DMA((NBUF,)),
        ],
    )
    def body(idx_hbm, pe_hbm, table_hbm, out_hbm,
             idx_v, pe_v, rows_v, idx_sem, pe_sem, gat_sem, st_sem):
        wid = lax.axis_index("s") * NUM_CORES + lax.axis_index("c")
        s_base = wid * s_per_worker

        def chunk_coords(c):
            sblock = c // (batch * subs)
            b = (c // subs) % batch
            sub = c % subs
            s0 = s_base + sblock * PE_ROWS + sub * CHUNK
            flat = b * seq_len + s0
            return sblock, sub, flat

        idx_descs = []
        for c in range(0, n_chunks, subs):
            _, _, flat = chunk_coords(c)
            idx_descs.append(pltpu.async_copy(
                idx_hbm.at[pl.ds(flat, PE_ROWS)],
                idx_v.at[pl.ds(c * CHUNK, PE_ROWS)], idx_sem))

        def fire_pe(sblock):
            return pltpu.async_copy(
                pe_hbm.at[pl.ds(s_base + sblock * PE_ROWS, PE_ROWS)],
                pe_v, pe_sem)

        pe_desc = fire_pe(0)
        for d in idx_descs:
            d.wait()

        def fire_gather(c):
            return pltpu.async_copy(
                table_hbm.at[idx_v.at[pl.ds(c * CHUNK, CHUNK)]],
                rows_v.at[c % NBUF], gat_sem.at[c % NBUF])

        SPLIT = 8
        PIECE = CHUNK // SPLIT

        def fire_store(c, p):
            _, _, flat = chunk_coords(c)
            return pltpu.async_copy(
                rows_v.at[c % NBUF, pl.ds(p * PIECE, PIECE)],
                out_hbm.at[pl.ds(flat + p * PIECE, PIECE)],
                st_sem.at[c % NBUF])

        LOOKAHEAD = NBUF - 2
        gat_descs = {}
        st_descs = {}
        for c in range(min(LOOKAHEAD, n_chunks)):
            gat_descs[c] = fire_gather(c)

        for c in range(n_chunks):
            nxt = c + LOOKAHEAD
            if nxt < n_chunks:
                if nxt >= NBUF:
                    for d in st_descs.pop(nxt - NBUF):
                        d.wait()
                gat_descs[nxt] = fire_gather(nxt)
            sblock, sub, _ = chunk_coords(c)
            if c == 0:
                pe_desc.wait()
            if sblock > 0 and c == sblock * batch * subs:
                pe_desc.wait()
            gat_descs.pop(c).wait()

            bb = c % NBUF
            pe_row0 = sub * CHUNK

            pieces = []
            for p in range(SPLIT):
                @plsc.parallel_loop(0, PIECE, 1)
                def add_row(r, bb=bb, r0=p * PIECE, pe_row0=pe_row0):
                    @plsc.parallel_loop(0, D_MODEL, LANES, unroll=8)
                    def _(o):
                        plsc.addupdate(
                            rows_v.at[bb, r0 + r, pl.ds(o, LANES)],
                            pe_v[pe_row0 + r0 + r, pl.ds(o, LANES)])
                pieces.append(fire_store(c, p))
            st_descs[c] = pieces

            last_of_sblock = (c + 1) % (batch * subs) == 0
            if last_of_sblock and sblock + 1 < n_sblocks:
                pe_desc = fire_pe(sblock + 1)

        for c in range(n_chunks - NBUF, n_chunks):
            for d in st_descs.pop(c):
                d.wait()

    return body(source_flat, pe, W_enc)


def kernel(source, target, W_enc):
    b, s = source.shape
    pe = _sinusoidal_pos_encoding(MAX_LEN, D_MODEL)[:s]
    out = _embed(source.reshape(-1), pe, W_enc, b, s)
    return out.reshape(b, s, D_MODEL)


# speedup vs baseline: 1.0703x; 1.0703x over previous
import functools

import jax
import jax.numpy as jnp
import numpy as np
from jax import lax
from jax.experimental import pallas as pl
from jax.experimental.pallas import tpu as pltpu
from jax.experimental.pallas import tpu_sc as plsc

D_MODEL = 768
MAX_LEN = 8192
LANES = 16

NUM_CORES = 2
NUM_SUBCORES = 16
NUM_WORKERS = NUM_CORES * NUM_SUBCORES

NBUF = 4
CHUNK = 32
PE_ROWS = 32


def _sinusoidal_pos_encoding(max_len, d_model):
    pos = np.arange(max_len, dtype=np.float32)[:, None]
    i = np.arange(0, d_model, 2, dtype=np.float32)
    angle = pos / np.power(10000.0, i / float(d_model))
    pe = np.zeros((max_len, d_model), dtype=np.float32)
    pe[:, 0::2] = np.sin(angle)
    pe[:, 1::2] = np.cos(angle)
    return jnp.asarray(pe)


@functools.partial(jax.jit, static_argnames=("batch", "seq_len"))
def _embed(source_flat, pe, W_enc, batch, seq_len):
    n = source_flat.shape[0]
    s_per_worker = seq_len // NUM_WORKERS
    n_sblocks = s_per_worker // PE_ROWS
    subs = PE_ROWS // CHUNK
    n_chunks = n_sblocks * batch * subs
    rows_per_worker = n // NUM_WORKERS
    mesh = plsc.VectorSubcoreMesh(
        core_axis_name="c", subcore_axis_name="s",
        num_cores=NUM_CORES, num_subcores=NUM_SUBCORES)

    @functools.partial(
        pl.kernel,
        mesh=mesh,
        out_type=jax.ShapeDtypeStruct((n, D_MODEL), jnp.float32),
        scratch_types=[
            pltpu.VMEM((rows_per_worker,), jnp.int32),
            pltpu.VMEM((PE_ROWS, D_MODEL), jnp.float32),
            pltpu.VMEM((NBUF, CHUNK, D_MODEL), jnp.float32),
            pltpu.SemaphoreType.DMA,
            pltpu.SemaphoreType.DMA,
            pltpu.SemaphoreType.DMA((NBUF,)),
            pltpu.SemaphoreType.DMA((NBUF,)),
        ],
    )
    def body(idx_hbm, pe_hbm, table_hbm, out_hbm,
             idx_v, pe_v, rows_v, idx_sem, pe_sem, gat_sem, st_sem):
        wid = lax.axis_index("s") * NUM_CORES + lax.axis_index("c")
        s_base = wid * s_per_worker

        def chunk_coords(c):
            sblock = c // (batch * subs)
            b = (c // subs) % batch
            sub = c % subs
            s0 = s_base + sblock * PE_ROWS + sub * CHUNK
            flat = b * seq_len + s0
            return sblock, sub, flat

        idx_descs = []
        for c in range(0, n_chunks, subs):
            _, _, flat = chunk_coords(c)
            idx_descs.append(pltpu.async_copy(
                idx_hbm.at[pl.ds(flat, PE_ROWS)],
                idx_v.at[pl.ds(c * CHUNK, PE_ROWS)], idx_sem))

        def fire_pe(sblock):
            return pltpu.async_copy(
                pe_hbm.at[pl.ds(s_base + sblock * PE_ROWS, PE_ROWS)],
                pe_v, pe_sem)

        pe_desc = fire_pe(0)
        for d in idx_descs:
            d.wait()

        def fire_gather(c):
            return pltpu.async_copy(
                table_hbm.at[idx_v.at[pl.ds(c * CHUNK, CHUNK)]],
                rows_v.at[c % NBUF], gat_sem.at[c % NBUF])

        SPLIT = 4
        PIECE = CHUNK // SPLIT

        def fire_store(c, p):
            _, _, flat = chunk_coords(c)
            return pltpu.async_copy(
                rows_v.at[c % NBUF, pl.ds(p * PIECE, PIECE)],
                out_hbm.at[pl.ds(flat + p * PIECE, PIECE)],
                st_sem.at[c % NBUF])

        LOOKAHEAD = NBUF - 2
        gat_descs = {}
        st_descs = {}
        for c in range(min(LOOKAHEAD, n_chunks)):
            gat_descs[c] = fire_gather(c)

        for c in range(n_chunks):
            nxt = c + LOOKAHEAD
            if nxt < n_chunks:
                if nxt >= NBUF:
                    for d in st_descs.pop(nxt - NBUF):
                        d.wait()
                gat_descs[nxt] = fire_gather(nxt)
            sblock, sub, _ = chunk_coords(c)
            if c == 0:
                pe_desc.wait()
            if sblock > 0 and c == sblock * batch * subs:
                pe_desc.wait()
            gat_descs.pop(c).wait()

            bb = c % NBUF
            pe_row0 = sub * CHUNK

            pieces = []
            for p in range(SPLIT):
                @plsc.parallel_loop(0, PIECE, 1)
                def add_row(r, bb=bb, r0=p * PIECE, pe_row0=pe_row0):
                    @plsc.parallel_loop(0, D_MODEL, LANES, unroll=8)
                    def _(o):
                        plsc.addupdate(
                            rows_v.at[bb, r0 + r, pl.ds(o, LANES)],
                            pe_v[pe_row0 + r0 + r, pl.ds(o, LANES)])
                pieces.append(fire_store(c, p))
            st_descs[c] = pieces

            last_of_sblock = (c + 1) % (batch * subs) == 0
            if last_of_sblock and sblock + 1 < n_sblocks:
                pe_desc = fire_pe(sblock + 1)

        for c in range(n_chunks - NBUF, n_chunks):
            for d in st_descs.pop(c):
                d.wait()

    return body(source_flat, pe, W_enc)


def kernel(source, target, W_enc):
    b, s = source.shape
    pe = _sinusoidal_pos_encoding(MAX_LEN, D_MODEL)[:s]
    out = _embed(source.reshape(-1), pe, W_enc, b, s)
    return out.reshape(b, s, D_MODEL)
